# Initial kernel scaffold; baseline (speedup 1.0000x reference)
import jax
import jax.numpy as jnp
from jax.experimental import pallas as pl


def kernel(x):
    raise NotImplementedError("write your pallas kernel here")


# speedup vs baseline: 10.7202x; 10.7202x over previous
import jax
import jax.numpy as jnp
import numpy as np
from jax.experimental import pallas as pl

_K = 512
_INT_MIN = np.int32(-2147483648)


def _topk_mask_kernel(x_ref, o_ref):
    x = x_ref[...]
    b = jax.lax.bitcast_convert_type(x, jnp.int32)
    key = jnp.where(b >= 0, b, jnp.bitwise_xor(jnp.invert(b), _INT_MIN))
    rows = x.shape[0]

    def body(i, prefix):
        shift = 31 - i
        cand = prefix + jnp.left_shift(np.int32(1), shift)
        cnt = jnp.sum((key >= cand).astype(jnp.float32), axis=1, keepdims=True)
        return jnp.where(cnt >= _K, cand, prefix)

    thresh = jax.lax.fori_loop(
        0, 32, body, jnp.full((rows, 1), _INT_MIN, jnp.int32)
    )
    o_ref[...] = jnp.where(key >= thresh, jnp.maximum(x, 0.0), 0.0)


def kernel(x):
    m, n = x.shape
    block_rows = 8
    return pl.pallas_call(
        _topk_mask_kernel,
        grid=(m // block_rows,),
        in_specs=[pl.BlockSpec((block_rows, n), lambda i: (i, 0))],
        out_specs=pl.BlockSpec((block_rows, n), lambda i: (i, 0)),
        out_shape=jax.ShapeDtypeStruct((m, n), x.dtype),
    )(x)


# speedup vs baseline: 11.1568x; 1.0407x over previous
import jax
import jax.numpy as jnp
import numpy as np
from jax.experimental import pallas as pl

_K = 512


def _topk_mask_kernel(x_ref, o_ref):
    x = x_ref[...]
    y = jnp.maximum(x, 0.0)
    key = jax.lax.bitcast_convert_type(y, jnp.int32)
    rows, n = y.shape
    key3 = key.reshape(rows, n // 128, 128)

    def body(i, prefix):
        shift = 30 - i
        cand = prefix + jnp.left_shift(np.int32(1), shift)
        m = (key3 >= cand[:, :, None]).astype(jnp.float32)
        part = jnp.sum(m, axis=1)
        cnt = jnp.sum(part, axis=1, keepdims=True)
        return jnp.where(cnt >= _K, cand, prefix)

    thresh = jax.lax.fori_loop(
        0, 31, body, jnp.zeros((rows, 1), jnp.int32)
    )
    o_ref[...] = jnp.where(key >= thresh, y, 0.0)


def kernel(x):
    m, n = x.shape
    block_rows = 8
    return pl.pallas_call(
        _topk_mask_kernel,
        grid=(m // block_rows,),
        in_specs=[pl.BlockSpec((block_rows, n), lambda i: (i, 0))],
        out_specs=pl.BlockSpec((block_rows, n), lambda i: (i, 0)),
        out_shape=jax.ShapeDtypeStruct((m, n), x.dtype),
    )(x)
